# Initial kernel scaffold; baseline (speedup 1.0000x reference)
import jax
import jax.numpy as jnp
from jax.experimental import pallas as pl


def kernel(coordinates, features, distsq, neighbour_indices):
    raise NotImplementedError("write your pallas kernel here")


# speedup vs baseline: 37.9684x; 37.9684x over previous
import functools

import jax
import jax.numpy as jnp
from jax import lax
from jax.experimental import pallas as pl
from jax.experimental.pallas import tpu as pltpu
from jax.experimental.pallas import tpu_sc as plsc

_V, _D, _F, _K = 10000, 4, 128, 32
_L = 16
_CH = 8
_NCH = _V // _CH
_NC, _NS = 2, 16
_NW = _NC * _NS
_NOFF = 9
_FC = _F // _L


def _splat_i32(x):
    return jnp.full((_L,), x, dtype=jnp.int32)


def _sc_body(coords_hbm, feats_hbm, nbr_hbm, out_hbm,
             coords_v, idx_v, rows_v, wbuf_v, outbuf_v, sem_g, sem_o):
    wid = lax.axis_index("s") * _NC + lax.axis_index("c")
    pltpu.sync_copy(coords_hbm, coords_v)
    nchunks = (_NCH - wid + _NW - 1) // _NW

    def chunk_body(i, carry):
        c = wid + i * _NW
        v0 = c * _CH
        pltpu.sync_copy(nbr_hbm.at[c], idx_v)
        cp0 = pltpu.async_copy(feats_hbm.at[idx_v.at[0]],
                               rows_v.at[pl.ds(0, 128)], sem_g)
        cp1 = pltpu.async_copy(feats_hbm.at[idx_v.at[1]],
                               rows_v.at[pl.ds(128, 128)], sem_g)

        for v in range(_CH):
            cc = [plsc.load_gather(coords_v, [_splat_i32((v0 + v) * _D + d)])
                  for d in range(_D)]
            for h in range(_K // _L):
                flat = v * _K + h * _L
                nidx = idx_v[flat // 128, pl.ds(flat % 128, _L)] * _D
                dsq = jnp.zeros((_L,), jnp.float32)
                for d in range(_D):
                    cn = plsc.load_gather(coords_v, [nidx + d])
                    df = cn - cc[d]
                    dsq = dsq + df * df
                wbuf_v[v, pl.ds(h * _L, _L)] = jnp.exp(dsq * -10.0) * (1.0 / _K)

        cp0.wait()
        cp1.wait()

        for v in range(_CH):
            def acc_body(k, acc):
                wk = plsc.load_gather(wbuf_v, [_splat_i32(v), _splat_i32(k)])
                row = v * _K + k
                return tuple(acc[j] + wk * rows_v[row, pl.ds(j * _L, _L)]
                             for j in range(_FC))

            acc0 = tuple(jnp.zeros((_L,), jnp.float32) for _ in range(_FC))
            acc = lax.fori_loop(0, _K, acc_body, acc0)
            for j in range(_FC):
                outbuf_v[v, pl.ds(j * _L, _L)] = acc[j]

        cps = [pltpu.async_copy(outbuf_v,
                                out_hbm.at[pl.ds(v0, _CH), pl.ds(o * _F, _F)],
                                sem_o)
               for o in range(_NOFF)]
        for cp in cps:
            cp.wait()
        return carry

    lax.fori_loop(0, nchunks, chunk_body, 0)


_sc_kernel = functools.partial(
    pl.kernel,
    out_type=jax.ShapeDtypeStruct((_V, _NOFF * _F), jnp.float32),
    mesh=plsc.VectorSubcoreMesh(core_axis_name="c", subcore_axis_name="s"),
    compiler_params=pltpu.CompilerParams(needs_layout_passes=False),
    scratch_types=[
        pltpu.VMEM((_V * _D,), jnp.float32),
        pltpu.VMEM((2, 128), jnp.int32),
        pltpu.VMEM((_CH * _K, _F), jnp.float32),
        pltpu.VMEM((_CH, _K), jnp.float32),
        pltpu.VMEM((_CH, _F), jnp.float32),
        pltpu.SemaphoreType.DMA,
        pltpu.SemaphoreType.DMA,
    ],
)(_sc_body)


@jax.jit
def kernel(coordinates, features, distsq, neighbour_indices):
    del distsq
    nbr = neighbour_indices.reshape(_NCH, 2, 128)
    return _sc_kernel(coordinates.reshape(-1), features, nbr)


# speedup vs baseline: 41.1928x; 1.0849x over previous
import functools

import jax
import jax.numpy as jnp
from jax import lax
from jax.experimental import pallas as pl
from jax.experimental.pallas import tpu as pltpu
from jax.experimental.pallas import tpu_sc as plsc

_V, _D, _F, _K = 10000, 4, 128, 32
_L = 16
_CH = 8
_NCH = _V // _CH
_NC, _NS = 2, 16
_NW = _NC * _NS
_NOFF = 9
_FC = _F // _L
_FO = _NOFF * _F


def _splat_i32(x):
    return jnp.full((_L,), x, dtype=jnp.int32)


def _sc_body(coords_hbm, feats_hbm, nbr_hbm, out_hbm,
             coords_v, idx_v, rows_v, wbuf_v, outbuf_v, sem_i, sem_g, sem_o):
    wid = lax.axis_index("s") * _NC + lax.axis_index("c")
    pltpu.sync_copy(coords_hbm, coords_v)
    n = (_NCH - wid + _NW - 1) // _NW

    def fire_idx(i, s):
        pltpu.async_copy(nbr_hbm.at[wid + i * _NW], idx_v.at[s], sem_i)

    fire_idx(0, 0)

    def body(i, carry):
        s = jnp.bitwise_and(i, 1)
        sp = jnp.bitwise_and(i + 1, 1)

        @pl.when(jnp.logical_and(i >= 1, i <= n))
        def _consume():
            cm1 = wid + (i - 1) * _NW
            v0 = cm1 * _CH
            for v in range(_CH):
                cc = [plsc.load_gather(coords_v,
                                       [_splat_i32((v0 + v) * _D + d)])
                      for d in range(_D)]
                for h in range(_K // _L):
                    flat = v * _K + h * _L
                    nidx = idx_v[sp, flat // 128, pl.ds(flat % 128, _L)] * _D
                    dsq = jnp.zeros((_L,), jnp.float32)
                    for d in range(_D):
                        df = plsc.load_gather(coords_v, [nidx + d]) - cc[d]
                        dsq = dsq + df * df
                    wbuf_v[v, pl.ds(h * _L, _L)] = (
                        jnp.exp(dsq * -10.0) * (1.0 / _K))

            pltpu.make_async_copy(feats_hbm.at[idx_v.at[sp, 0]],
                                  rows_v.at[sp, pl.ds(0, 128)], sem_g).wait()
            pltpu.make_async_copy(feats_hbm.at[idx_v.at[sp, 1]],
                                  rows_v.at[sp, pl.ds(128, 128)], sem_g).wait()

            @pl.when(i >= 3)
            def _():
                pltpu.make_async_copy(outbuf_v.at[sp],
                                      out_hbm.at[pl.ds(0, _CH)], sem_o).wait()

            for v in range(_CH):
                def acc_body(k, acc):
                    wk = plsc.load_gather(wbuf_v,
                                          [_splat_i32(v), _splat_i32(k)])
                    row = v * _K + k
                    return tuple(acc[j] + wk * rows_v[sp, row,
                                                      pl.ds(j * _L, _L)]
                                 for j in range(_FC))

                acc0 = tuple(jnp.zeros((_L,), jnp.float32)
                             for _ in range(_FC))
                acc = lax.fori_loop(0, _K, acc_body, acc0)
                for j in range(_FC):
                    for o in range(_NOFF):
                        outbuf_v[sp, v, pl.ds(o * _F + j * _L, _L)] = acc[j]

            pltpu.async_copy(outbuf_v.at[sp],
                             out_hbm.at[pl.ds(v0, _CH)], sem_o)

        @pl.when(i < n)
        def _fire_gathers():
            pltpu.make_async_copy(nbr_hbm.at[wid],
                                  idx_v.at[s], sem_i).wait()
            pltpu.async_copy(feats_hbm.at[idx_v.at[s, 0]],
                             rows_v.at[s, pl.ds(0, 128)], sem_g)
            pltpu.async_copy(feats_hbm.at[idx_v.at[s, 1]],
                             rows_v.at[s, pl.ds(128, 128)], sem_g)

        @pl.when(i + 1 < n)
        def _prefetch_idx():
            fire_idx(i + 1, sp)

        return carry

    lax.fori_loop(0, n + 1, body, 0)

    for _ in range(2):
        pltpu.make_async_copy(outbuf_v.at[0],
                              out_hbm.at[pl.ds(0, _CH)], sem_o).wait()


_sc_kernel = functools.partial(
    pl.kernel,
    out_type=jax.ShapeDtypeStruct((_V, _FO), jnp.float32),
    mesh=plsc.VectorSubcoreMesh(core_axis_name="c", subcore_axis_name="s"),
    compiler_params=pltpu.CompilerParams(needs_layout_passes=False),
    scratch_types=[
        pltpu.VMEM((_V * _D,), jnp.float32),
        pltpu.VMEM((2, 2, 128), jnp.int32),
        pltpu.VMEM((2, _CH * _K, _F), jnp.float32),
        pltpu.VMEM((_CH, _K), jnp.float32),
        pltpu.VMEM((2, _CH, _FO), jnp.float32),
        pltpu.SemaphoreType.DMA,
        pltpu.SemaphoreType.DMA,
        pltpu.SemaphoreType.DMA,
    ],
)(_sc_body)


@jax.jit
def kernel(coordinates, features, distsq, neighbour_indices):
    del distsq
    nbr = neighbour_indices.reshape(_NCH, 2, 128)
    return _sc_kernel(coordinates.reshape(-1), features, nbr)
